# initial kernel scaffold (unmeasured)
import jax
import jax.numpy as jnp
from jax import lax
from jax.experimental import pallas as pl
from jax.experimental.pallas import tpu as pltpu

N_DEV = 32

_DEV_ID_TYPE = getattr(pl, "DeviceIdType", None) or pltpu.DeviceIdType


def kernel(x, w_mat):
    m_per, k = x.shape
    _, n = w_mat.shape
    n_per = n // N_DEV
    pad = 8
    ch = m_per + pad

    def body(x_ref, w_ref, out_ref, send_ref, recv_ref, send_sems, recv_sems):
        my = lax.axis_index("i")

        y = jnp.dot(x_ref[...], w_ref[...], preferred_element_type=jnp.float32)
        amax = jnp.max(jnp.abs(y))

        for j in range(N_DEV):
            send_ref[j, 0:m_per, :] = y[:, j * n_per:(j + 1) * n_per]
        send_ref[:, m_per:ch, :] = jnp.full((N_DEV, pad, n_per), amax,
                                            jnp.float32)

        recv_ref[my] = send_ref[my]

        sends = []
        for off in range(1, N_DEV):
            tgt = lax.rem(my + off, N_DEV)
            rdma = pltpu.make_async_remote_copy(
                src_ref=send_ref.at[tgt],
                dst_ref=recv_ref.at[my],
                send_sem=send_sems.at[tgt],
                recv_sem=recv_sems.at[my],
                device_id=(tgt,),
                device_id_type=_DEV_ID_TYPE.MESH,
            )
            rdma.start()
            sends.append(rdma)

        for off in range(1, N_DEV):
            src = lax.rem(my + N_DEV - off, N_DEV)
            recv = pltpu.make_async_remote_copy(
                src_ref=send_ref.at[src],
                dst_ref=recv_ref.at[src],
                send_sem=send_sems.at[src],
                recv_sem=recv_sems.at[src],
                device_id=(src,),
                device_id_type=_DEV_ID_TYPE.MESH,
            )
            recv.wait_recv()

        gmax = jnp.max(recv_ref[:, m_per, :])
        scale = gmax / 127.0
        data = recv_ref[:, 0:m_per, :].reshape(N_DEV * m_per, n_per)
        q = jnp.clip(jnp.round(data / scale), -127.0, 127.0)
        out_ref[...] = q * scale

        for rdma in sends:
            rdma.wait_send()

    return pl.pallas_call(
        body,
        out_shape=jax.ShapeDtypeStruct((N_DEV * m_per, n_per), jnp.float32),
        in_specs=[
            pl.BlockSpec(memory_space=pltpu.VMEM),
            pl.BlockSpec(memory_space=pltpu.VMEM),
        ],
        out_specs=pl.BlockSpec(memory_space=pltpu.VMEM),
        scratch_shapes=[
            pltpu.VMEM((N_DEV, ch, n_per), jnp.float32),
            pltpu.VMEM((N_DEV, ch, n_per), jnp.float32),
            pltpu.SemaphoreType.DMA((N_DEV,)),
            pltpu.SemaphoreType.DMA((N_DEV,)),
        ],
    )(x, w_mat)


# baseline (device time: 57914 ns/iter reference)
import jax
import jax.numpy as jnp
from jax import lax
from jax.experimental import pallas as pl
from jax.experimental.pallas import tpu as pltpu

N_DEV = 32

_DEV_ID_TYPE = getattr(pl, "DeviceIdType", None) or pltpu.DeviceIdType
_COMPILER_PARAMS = getattr(pltpu, "CompilerParams", None) or pltpu.TPUCompilerParams


def kernel(x, w_mat):
    m_per, k = x.shape
    _, n = w_mat.shape
    n_per = n // N_DEV
    pad = 8
    ch = m_per + pad

    def body(x_ref, w_ref, out_ref, send_ref, recv_ref, send_sems, recv_sems):
        my = lax.axis_index("i")

        y = jnp.dot(x_ref[...], w_ref[...], preferred_element_type=jnp.float32)
        amax = jnp.max(jnp.abs(y))

        for j in range(N_DEV):
            send_ref[j, 0:m_per, :] = y[:, j * n_per:(j + 1) * n_per]
        send_ref[:, m_per:ch, :] = jnp.full((N_DEV, pad, n_per), amax,
                                            jnp.float32)

        recv_ref[my] = send_ref[my]

        sends = []
        for off in range(1, N_DEV):
            tgt = lax.rem(my + off, N_DEV)
            rdma = pltpu.make_async_remote_copy(
                src_ref=send_ref.at[tgt],
                dst_ref=recv_ref.at[my],
                send_sem=send_sems.at[tgt],
                recv_sem=recv_sems.at[my],
                device_id=(tgt,),
                device_id_type=_DEV_ID_TYPE.MESH,
            )
            rdma.start()
            sends.append(rdma)

        for off in range(1, N_DEV):
            src = lax.rem(my + N_DEV - off, N_DEV)
            recv = pltpu.make_async_remote_copy(
                src_ref=send_ref.at[src],
                dst_ref=recv_ref.at[src],
                send_sem=send_sems.at[src],
                recv_sem=recv_sems.at[src],
                device_id=(src,),
                device_id_type=_DEV_ID_TYPE.MESH,
            )
            recv.wait_recv()

        gmax = jnp.max(recv_ref[:, m_per, :])
        scale = gmax / 127.0
        data = recv_ref[:, 0:m_per, :].reshape(N_DEV * m_per, n_per)
        q = jnp.clip(jnp.round(data / scale), -127.0, 127.0)
        out_ref[...] = q * scale

        for rdma in sends:
            rdma.wait_send()

    return pl.pallas_call(
        body,
        out_shape=jax.ShapeDtypeStruct((N_DEV * m_per, n_per), jnp.float32),
        in_specs=[
            pl.BlockSpec(memory_space=pltpu.VMEM),
            pl.BlockSpec(memory_space=pltpu.VMEM),
        ],
        out_specs=pl.BlockSpec(memory_space=pltpu.VMEM),
        scratch_shapes=[
            pltpu.VMEM((N_DEV, ch, n_per), jnp.float32),
            pltpu.VMEM((N_DEV, ch, n_per), jnp.float32),
            pltpu.SemaphoreType.DMA((N_DEV,)),
            pltpu.SemaphoreType.DMA((N_DEV,)),
        ],
        compiler_params=_COMPILER_PARAMS(vmem_limit_bytes=60 * 1024 * 1024),
    )(x, w_mat)
